# baseline (speedup 1.0000x reference)
import math

import jax
import jax.numpy as jnp
from jax.experimental import pallas as pl
from jax.experimental.pallas import tpu as pltpu

B = 32
T = 12
N = 100
NP = 104
D = 64
GRU = 64
QK = 32
HOR = 3
NB = 3
TD = T * D
BNP = B * NP
NH = N * HOR
KTOT = T * N * D
NG = 5
KC = NG * TD
NCH = N // NG
KSIZES = (3, 5, 7)
RC = 128


def _fused_kernel(xTT_ref, wih_ref, whh_ref, bih_ref, bhh_ref,
                  wqT_ref, wqb_ref, wkT_ref, wkb_ref,
                  c1w_ref, c1b_ref, gcnwT_ref, wr0_ref, wr1_ref, wr2_ref,
                  bias_ref, lmask_ref, rmask_ref, linw_ref, linb_ref,
                  out_ref, bufF, bufG, h3, wsm_ref, m_ref, h_s, gh_s):
    f32 = jnp.float32
    i = pl.program_id(0)

    @pl.when(i == 0)
    def _dense():
        wih = wih_ref[...]
        bih = bih_ref[...]
        whh = whh_ref[...]
        bhh = bhh_ref[...]
        h_s[...] = jnp.zeros((BNP, GRU), f32)
        for t in range(T):
            gh_s[...] = jax.lax.dot_general(
                h_s[...], whh, (((1,), (1,)), ((), ())),
                preferred_element_type=f32) + bhh
            for c in range(BNP // RC):
                r0 = c * RC
                ghc = gh_s[r0:r0 + RC, :]
                gic = xTT_ref[r0:r0 + RC, t:t + 1] * wih + bih
                r = jax.nn.sigmoid(gic[:, :GRU] + ghc[:, :GRU])
                z = jax.nn.sigmoid(gic[:, GRU:2 * GRU] + ghc[:, GRU:2 * GRU])
                n = jnp.tanh(gic[:, 2 * GRU:] + r * ghc[:, 2 * GRU:])
                h_s[r0:r0 + RC, :] = (1.0 - z) * n + z * h_s[r0:r0 + RC, :]
        h = h_s[...]

        q = (jnp.dot(h, wqT_ref[...], preferred_element_type=f32)
             + wqb_ref[...])
        k = (jnp.dot(h, wkT_ref[...], preferred_element_type=f32)
             + wkb_ref[...])
        scale = 1.0 / math.sqrt(QK)
        lmask = lmask_ref[...]
        for g in range(B):
            qg = q[g * NP:(g + 1) * NP, :]
            kg = k[g * NP:(g + 1) * NP, :]
            s = jax.lax.dot_general(qg, kg, (((1,), (1,)), ((), ())),
                                    preferred_element_type=f32) * scale
            wsm_ref[g * NP:(g + 1) * NP, :] = s + lmask
        sall = wsm_ref[...]
        sall = sall - jnp.max(sall, axis=1, keepdims=True)
        e = jnp.exp(sall)
        wsm_ref[...] = e / jnp.sum(e, axis=1, keepdims=True)

        rmask = rmask_ref[...]
        for g in range(B):
            sl = slice(g * NP, (g + 1) * NP)
            wg = wsm_ref[sl, :]
            degrow = jax.lax.dot_general(wg, rmask, (((0,), (0,)), ((), ())),
                                         preferred_element_type=f32)
            deglane = jax.lax.dot_general(rmask, wg, (((0,), (0,)), ((), ())),
                                          preferred_element_type=f32)
            disrow = jnp.where(degrow > 0.0,
                               jax.lax.rsqrt(jnp.where(degrow > 0.0,
                                                       degrow, 1.0)), 0.0)
            dislane = jnp.where(deglane > 0.0,
                                jax.lax.rsqrt(jnp.where(deglane > 0.0,
                                                        deglane, 1.0)), 0.0)
            wsm_ref[sl, :] = wg * disrow * dislane

        c1w = c1w_ref[...]
        c1b = c1b_ref[...]
        for t in range(T):
            xt = xTT_ref[:, t:t + 1]
            bufF[:, t * D:(t + 1) * D] = xt * c1w + c1b

        wrow_refs = (wr0_ref, wr1_ref, wr2_ref)
        for b in range(NB):
            ksz = KSIZES[b]
            p = ksz // 2
            wrow = wrow_refs[b][...]
            m_ref[...] = jnp.zeros((TD, TD), f32)
            for tau in range(T):
                lo = max(0, tau - p)
                hi = min(T, tau + p + 1)
                wsl = wrow[:, (lo - tau + p) * D:(hi - tau + p) * D]
                m_ref[tau * D:(tau + 1) * D, lo * D:hi * D] = jnp.dot(
                    gcnwT_ref[b, tau], wsl, preferred_element_type=f32)
            bias = bias_ref[b]
            bufG[...] = jnp.dot(bufF[...], m_ref[...],
                                preferred_element_type=f32)
            for g in range(B):
                sl = slice(g * NP, (g + 1) * NP)
                o = jax.lax.dot_general(wsm_ref[sl, :], bufG[sl, :],
                                        (((0,), (0,)), ((), ())),
                                        preferred_element_type=f32) + bias
                o = jnp.where(o >= 0.0, o, 0.01 * o)
                if b == NB - 1:
                    h3[g] = o
                else:
                    bufF[sl, :] = o

        out_ref[...] = jnp.zeros((B, NH), f32) + linb_ref[...]

    @pl.when(i > 0)
    def _lin():
        base = (i - 1) * NG
        xch = jnp.concatenate(
            [h3[:, pl.ds(base + nn, 1), :].reshape(B, TD)
             for nn in range(NG)], axis=1)
        out_ref[...] += jax.lax.dot_general(
            xch, linw_ref[...], (((1,), (1,)), ((), ())),
            preferred_element_type=f32)


def _toeplitz(conv_w, ksz):
    p = ksz // 2
    taps = jnp.transpose(conv_w, (2, 1, 0))
    taps_ext = jnp.concatenate(
        [taps, jnp.zeros((D, D), jnp.float32)[None]], axis=0)
    tau = jnp.arange(T)[:, None]
    t = jnp.arange(T)[None, :]
    idx = tau - t + p
    idx = jnp.where((idx >= 0) & (idx < ksz), idx, ksz)
    w4 = taps_ext[idx]
    return jnp.transpose(w4, (0, 2, 1, 3)).reshape(TD, TD)


def kernel(x, c1_w, c1_b, gru_wih, gru_whh, gru_bih, gru_bhh,
           wq_w, wq_b, wk_w, wk_b, gcn_w, gcn_b,
           conv_w0, conv_b0, conv_w1, conv_b1, conv_w2, conv_b2,
           lin_w, lin_b):
    f32 = jnp.float32
    x3 = jnp.transpose(x, (0, 2, 1))
    x3 = jnp.pad(x3, ((0, 0), (0, NP - N), (0, 0)))
    xTT = x3.reshape(BNP, T)
    wihT = gru_wih.reshape(1, 3 * GRU)
    bih2 = gru_bih.reshape(1, 3 * GRU)
    bhh2 = gru_bhh.reshape(1, 3 * GRU)
    wqT = wq_w.T
    wkT = wk_w.T
    wqb2 = wq_b.reshape(1, QK)
    wkb2 = wk_b.reshape(1, QK)
    c1w2 = c1_w.reshape(1, D)
    c1b2 = c1_b.reshape(1, D)
    gcn_wT = jnp.swapaxes(gcn_w, 2, 3)
    wrow0 = jnp.transpose(conv_w0, (1, 2, 0))[:, ::-1, :].reshape(D, 3 * D)
    wrow1 = jnp.transpose(conv_w1, (1, 2, 0))[:, ::-1, :].reshape(D, 5 * D)
    wrow2 = jnp.transpose(conv_w2, (1, 2, 0))[:, ::-1, :].reshape(D, 7 * D)
    biasrows = jnp.stack([
        gcn_b[b].reshape(1, TD) @ _toeplitz((conv_w0, conv_w1, conv_w2)[b],
                                            KSIZES[b])
        + jnp.tile((conv_b0, conv_b1, conv_b2)[b], T)[None]
        for b in range(NB)])
    lane = jnp.arange(NP)
    lmask = jnp.where(lane < N, 0.0, -1e30).reshape(1, NP).astype(f32)
    rmask = jnp.where(lane < N, 1.0, 0.0).reshape(NP, 1).astype(f32)

    vmem = pl.BlockSpec(memory_space=pltpu.MemorySpace.VMEM)
    out = pl.pallas_call(
        _fused_kernel,
        grid=(NCH + 1,),
        in_specs=[vmem] * 18
        + [pl.BlockSpec((NH, KC), lambda i: (0, jnp.maximum(i - 1, 0))),
           vmem],
        out_specs=pl.BlockSpec((B, NH), lambda i: (0, 0)),
        out_shape=jax.ShapeDtypeStruct((B, NH), f32),
        scratch_shapes=[
            pltpu.VMEM((BNP, TD), f32),
            pltpu.VMEM((BNP, TD), f32),
            pltpu.VMEM((B, NP, TD), f32),
            pltpu.VMEM((BNP, NP), f32),
            pltpu.VMEM((TD, TD), f32),
            pltpu.VMEM((BNP, GRU), f32),
            pltpu.VMEM((BNP, 3 * GRU), f32),
        ],
    )(xTT, wihT, gru_whh, bih2, bhh2, wqT, wqb2, wkT, wkb2,
      c1w2, c1b2, gcn_wT, wrow0, wrow1, wrow2, biasrows, lmask, rmask,
      lin_w, lin_b.reshape(1, NH))
    return out


# speedup vs baseline: 1.2578x; 1.2578x over previous
import math

import jax
import jax.numpy as jnp
from jax.experimental import pallas as pl
from jax.experimental.pallas import tpu as pltpu

B = 32
T = 12
N = 100
NP = 104
D = 64
GRU = 64
QK = 32
HOR = 3
NB = 3
TD = T * D
BNP = B * NP
NH = N * HOR
KTOT = T * N * D
NG = 5
KC = NG * TD
NCH = N // NG
KSIZES = (3, 5, 7)
RC = 128


def _fused_kernel(xTT_ref, wih_ref, whh_ref, bih_ref, bhh_ref,
                  wqT_ref, wqb_ref, wkT_ref, wkb_ref,
                  c1w_ref, c1b_ref, gcnwT_ref, wr0_ref, wr1_ref, wr2_ref,
                  bias_ref, lmask_ref, rmask_ref,
                  hout_ref, bufF, bufG, wsm_ref, m_ref):
    f32 = jnp.float32
    if True:
        wih = wih_ref[...]
        bih = bih_ref[...]
        whh = whh_ref[...]
        bhh = bhh_ref[...]
        h = jnp.zeros((BNP, GRU), f32)
        for t in range(T):
            xt = xTT_ref[:, t:t + 1]
            gi = xt * wih + bih
            gh = jax.lax.dot_general(h, whh, (((1,), (1,)), ((), ())),
                                     preferred_element_type=f32) + bhh
            r = jax.nn.sigmoid(gi[:, :GRU] + gh[:, :GRU])
            z = jax.nn.sigmoid(gi[:, GRU:2 * GRU] + gh[:, GRU:2 * GRU])
            n = jnp.tanh(gi[:, 2 * GRU:] + r * gh[:, 2 * GRU:])
            h = (1.0 - z) * n + z * h

        q = (jnp.dot(h, wqT_ref[...], preferred_element_type=f32)
             + wqb_ref[...])
        k = (jnp.dot(h, wkT_ref[...], preferred_element_type=f32)
             + wkb_ref[...])
        scale = 1.0 / math.sqrt(QK)
        lmask = lmask_ref[...]
        for g in range(B):
            qg = q[g * NP:(g + 1) * NP, :]
            kg = k[g * NP:(g + 1) * NP, :]
            s = jax.lax.dot_general(qg, kg, (((1,), (1,)), ((), ())),
                                    preferred_element_type=f32) * scale
            wsm_ref[g * NP:(g + 1) * NP, :] = s + lmask
        sall = wsm_ref[...]
        sall = sall - jnp.max(sall, axis=1, keepdims=True)
        e = jnp.exp(sall)
        wsm_ref[...] = e / jnp.sum(e, axis=1, keepdims=True)

        rmask = rmask_ref[...]
        for g in range(B):
            sl = slice(g * NP, (g + 1) * NP)
            wg = wsm_ref[sl, :]
            degrow = jax.lax.dot_general(wg, rmask, (((0,), (0,)), ((), ())),
                                         preferred_element_type=f32)
            deglane = jax.lax.dot_general(rmask, wg, (((0,), (0,)), ((), ())),
                                          preferred_element_type=f32)
            disrow = jnp.where(degrow > 0.0,
                               jax.lax.rsqrt(jnp.where(degrow > 0.0,
                                                       degrow, 1.0)), 0.0)
            dislane = jnp.where(deglane > 0.0,
                                jax.lax.rsqrt(jnp.where(deglane > 0.0,
                                                        deglane, 1.0)), 0.0)
            wsm_ref[sl, :] = wg * disrow * dislane

        c1w = c1w_ref[...]
        c1b = c1b_ref[...]
        for t in range(T):
            xt = xTT_ref[:, t:t + 1]
            bufF[:, t * D:(t + 1) * D] = xt * c1w + c1b

        wrow_refs = (wr0_ref, wr1_ref, wr2_ref)
        for b in range(NB):
            ksz = KSIZES[b]
            p = ksz // 2
            wrow = wrow_refs[b][...]
            m_ref[...] = jnp.zeros((TD, TD), f32)
            for tau in range(T):
                lo = max(0, tau - p)
                hi = min(T, tau + p + 1)
                wsl = wrow[:, (lo - tau + p) * D:(hi - tau + p) * D]
                m_ref[tau * D:(tau + 1) * D, lo * D:hi * D] = jnp.dot(
                    gcnwT_ref[b, tau], wsl, preferred_element_type=f32)
            bias = bias_ref[b]
            bufG[...] = jnp.dot(bufF[...], m_ref[...],
                                preferred_element_type=f32)
            for g in range(B):
                sl = slice(g * NP, (g + 1) * NP)
                o = jax.lax.dot_general(wsm_ref[sl, :], bufG[sl, :],
                                        (((0,), (0,)), ((), ())),
                                        preferred_element_type=f32) + bias
                o = jnp.where(o >= 0.0, o, 0.01 * o)
                if b == NB - 1:
                    hout_ref[g] = o[:N, :]
                else:
                    bufF[sl, :] = o


def _lin_kernel(x_ref, w_ref, b_ref, o_ref):
    i = pl.program_id(0)
    part = jax.lax.dot_general(x_ref[...], w_ref[...],
                               (((1,), (1,)), ((), ())),
                               preferred_element_type=jnp.float32)

    @pl.when(i == 0)
    def _init():
        o_ref[...] = part + b_ref[...]

    @pl.when(i > 0)
    def _acc():
        o_ref[...] += part


def _toeplitz(conv_w, ksz):
    p = ksz // 2
    taps = jnp.transpose(conv_w, (2, 1, 0))
    taps_ext = jnp.concatenate(
        [taps, jnp.zeros((D, D), jnp.float32)[None]], axis=0)
    tau = jnp.arange(T)[:, None]
    t = jnp.arange(T)[None, :]
    idx = tau - t + p
    idx = jnp.where((idx >= 0) & (idx < ksz), idx, ksz)
    w4 = taps_ext[idx]
    return jnp.transpose(w4, (0, 2, 1, 3)).reshape(TD, TD)


def kernel(x, c1_w, c1_b, gru_wih, gru_whh, gru_bih, gru_bhh,
           wq_w, wq_b, wk_w, wk_b, gcn_w, gcn_b,
           conv_w0, conv_b0, conv_w1, conv_b1, conv_w2, conv_b2,
           lin_w, lin_b):
    f32 = jnp.float32
    x3 = jnp.transpose(x, (0, 2, 1))
    x3 = jnp.pad(x3, ((0, 0), (0, NP - N), (0, 0)))
    xTT = x3.reshape(BNP, T)
    wihT = gru_wih.reshape(1, 3 * GRU)
    bih2 = gru_bih.reshape(1, 3 * GRU)
    bhh2 = gru_bhh.reshape(1, 3 * GRU)
    wqT = wq_w.T
    wkT = wk_w.T
    wqb2 = wq_b.reshape(1, QK)
    wkb2 = wk_b.reshape(1, QK)
    c1w2 = c1_w.reshape(1, D)
    c1b2 = c1_b.reshape(1, D)
    gcn_wT = jnp.swapaxes(gcn_w, 2, 3)
    wrow0 = jnp.transpose(conv_w0, (1, 2, 0))[:, ::-1, :].reshape(D, 3 * D)
    wrow1 = jnp.transpose(conv_w1, (1, 2, 0))[:, ::-1, :].reshape(D, 5 * D)
    wrow2 = jnp.transpose(conv_w2, (1, 2, 0))[:, ::-1, :].reshape(D, 7 * D)
    biasrows = jnp.stack([
        gcn_b[b].reshape(1, TD) @ _toeplitz((conv_w0, conv_w1, conv_w2)[b],
                                            KSIZES[b])
        + jnp.tile((conv_b0, conv_b1, conv_b2)[b], T)[None]
        for b in range(NB)])
    lane = jnp.arange(NP)
    lmask = jnp.where(lane < N, 0.0, -1e30).reshape(1, NP).astype(f32)
    rmask = jnp.where(lane < N, 1.0, 0.0).reshape(NP, 1).astype(f32)

    hfull = pl.pallas_call(
        _fused_kernel,
        out_shape=jax.ShapeDtypeStruct((B, N, TD), f32),
        scratch_shapes=[
            pltpu.VMEM((BNP, TD), f32),
            pltpu.VMEM((BNP, TD), f32),
            pltpu.VMEM((BNP, NP), f32),
            pltpu.VMEM((TD, TD), f32),
        ],
    )(xTT, wihT, gru_whh, bih2, bhh2, wqT, wqb2, wkT, wkb2,
      c1w2, c1b2, gcn_wT, wrow0, wrow1, wrow2, biasrows, lmask, rmask)

    xout = hfull.reshape(B, KTOT)
    nk = 12
    kc = KTOT // nk
    out = pl.pallas_call(
        _lin_kernel,
        grid=(nk,),
        in_specs=[
            pl.BlockSpec((B, kc), lambda i: (0, i)),
            pl.BlockSpec((NH, kc), lambda i: (0, i)),
            pl.BlockSpec((1, NH), lambda i: (0, 0)),
        ],
        out_specs=pl.BlockSpec((B, NH), lambda i: (0, 0)),
        out_shape=jax.ShapeDtypeStruct((B, NH), f32),
    )(xout, lin_w, lin_b.reshape(1, NH))
    return out
